# baseline (speedup 1.0000x reference)
import functools

import numpy as np
import jax
import jax.numpy as jnp
from jax import lax
from jax.experimental import pallas as pl
from jax.experimental.pallas import tpu as pltpu
from jax.experimental.pallas import tpu_sc as plsc

N = 10000
E = 320000
G = 100
NC, NS, LANES = 2, 16, 16
NW = NC * NS
CH = 125
CPT = E // NW // CH
NBUF = 8
INFLT = 4
NACC = 10240
RPT = NACC // NS
NP = 10240
PRT = NP // NW
GP = 128

_MESH = plsc.VectorSubcoreMesh(
    core_axis_name="c", subcore_axis_name="s", num_cores=NC, num_subcores=NS
)

_SC_PARAMS = pltpu.CompilerParams(use_tc_tiling_on_sc=False,
                                  needs_layout_passes=False)
_TC_PARAMS = pltpu.CompilerParams(vmem_limit_bytes=100 * 1024 * 1024)


def _dotT(a, b):
    return lax.dot_general(a, b, (((1,), (1,)), ((), ())),
                           preferred_element_type=jnp.float32)


def _edge_agg(table, edges3d, zeros, width):
    @functools.partial(
        pl.kernel,
        out_type=jax.ShapeDtypeStruct((NC, NACC, width), jnp.float32),
        mesh=_MESH,
        scratch_types=[
            pltpu.VMEM((CPT, CH), jnp.int32),
            pltpu.VMEM((CPT, CH), jnp.int32),
        ] + [pltpu.VMEM((CH, width), jnp.float32) for _ in range(NBUF)]
          + [pltpu.VMEM_SHARED((NACC, width), jnp.float32)]
          + [pltpu.SemaphoreType.DMA for _ in range(2 * NBUF)],
        compiler_params=_SC_PARAMS,
    )
    def k(table_h, edges_h, zeros_h, out_h, src_v, dst_v, *rest):
        bufs = rest[:NBUF]
        acc = rest[NBUF]
        gsem = rest[NBUF + 1:2 * NBUF + 1]
        ssem = rest[2 * NBUF + 1:]
        c = lax.axis_index("c")
        s = lax.axis_index("s")
        wid = c * NS + s
        pltpu.sync_copy(zeros_h.at[pl.ds(s * RPT, RPT)],
                        acc.at[pl.ds(s * RPT, RPT)])
        pltpu.sync_copy(edges_h.at[0, pl.ds(wid * CPT, CPT)], src_v)
        pltpu.sync_copy(edges_h.at[1, pl.ds(wid * CPT, CPT)], dst_v)
        for b in range(INFLT):
            pltpu.async_copy(table_h.at[src_v.at[b]], bufs[b], gsem[b])
        plsc.subcore_barrier()

        @pl.loop(0, CPT, step=NBUF)
        def _(g):
            for b in range(NBUF):
                j = g + b
                pltpu.make_async_copy(table_h.at[src_v.at[j]], bufs[b],
                                      gsem[b]).wait()
                pltpu.async_copy(bufs[b], acc.at[dst_v.at[j]], ssem[b],
                                 add=True)
                nxt = j + INFLT
                bb = (b + INFLT) % NBUF

                @pl.when(nxt < CPT)
                def _():
                    @pl.when(j >= INFLT)
                    def _():
                        pltpu.make_async_copy(
                            bufs[bb], acc.at[dst_v.at[b]], ssem[bb]).wait()

                    pltpu.async_copy(table_h.at[src_v.at[nxt]], bufs[bb],
                                     gsem[bb])

        for b in range(NBUF):
            pltpu.make_async_copy(bufs[b], acc.at[dst_v.at[b]],
                                  ssem[b]).wait()
        plsc.subcore_barrier()
        pltpu.sync_copy(acc.at[pl.ds(s * RPT, RPT)],
                        out_h.at[c, pl.ds(s * RPT, RPT)])

    return k(table, edges3d, zeros)


def _pool_sum(zsum_p, gid3d, zeros_pool):
    @functools.partial(
        pl.kernel,
        out_type=jax.ShapeDtypeStruct((NC, GP, 64), jnp.float32),
        mesh=_MESH,
        scratch_types=[
            pltpu.VMEM((1, PRT // 80, 80), jnp.int32),
            pltpu.VMEM((PRT, 64), jnp.float32),
            pltpu.VMEM_SHARED((GP, 64), jnp.float32),
        ],
        compiler_params=_SC_PARAMS,
    )
    def k(zsum_h, gid_h, zeros_h, outp_h, gv, zbuf, pacc):
        c = lax.axis_index("c")
        s = lax.axis_index("s")
        wid = c * NS + s

        @pl.when(s == 0)
        def _():
            pltpu.sync_copy(zeros_h, pacc)

        pltpu.sync_copy(gid_h.at[pl.ds(wid, 1)], gv)
        pltpu.sync_copy(zsum_h.at[pl.ds(wid * PRT, PRT)], zbuf)
        plsc.subcore_barrier()
        for j in range(PRT // 80):
            pltpu.sync_copy(zbuf.at[pl.ds(j * 80, 80)], pacc.at[gv.at[0, j]],
                            add=True)
        plsc.subcore_barrier()

        @pl.when(s == 0)
        def _():
            pltpu.sync_copy(pacc, outp_h.at[c])

    return k(zsum_p, gid3d, zeros_pool)


def _pool_kin(kin_p, gid3d):
    @functools.partial(
        pl.kernel,
        out_type=jax.ShapeDtypeStruct((NW, GP, 16), jnp.float32),
        mesh=_MESH,
        scratch_types=[
            pltpu.VMEM((1, PRT // 80, 80), jnp.int32),
            pltpu.VMEM((PRT, 16), jnp.float32),
            pltpu.VMEM((GP, 16), jnp.float32),
        ],
        compiler_params=_SC_PARAMS,
    )
    def k(kin_h, gid_h, outk_h, gv, kbuf, kout):
        c = lax.axis_index("c")
        s = lax.axis_index("s")
        wid = c * NS + s
        pltpu.sync_copy(gid_h.at[pl.ds(wid, 1)], gv)
        pltpu.sync_copy(kin_h.at[pl.ds(wid * PRT, PRT)], kbuf)

        lanes = lax.iota(jnp.int32, LANES)
        neg = jnp.full((LANES,), -jnp.inf, jnp.float32)

        @pl.loop(0, GP)
        def _(i):
            plsc.store_scatter(kout, [jnp.full((LANES,), i, jnp.int32),
                                      lanes], neg)

        @pl.loop(0, PRT)
        def _(r):
            gvec = plsc.load_gather(
                gv, [jnp.zeros((LANES,), jnp.int32),
                     jnp.full((LANES,), r // 80, jnp.int32),
                     jnp.full((LANES,), r % 80, jnp.int32)])
            v = plsc.load_gather(kbuf, [jnp.full((LANES,), r, jnp.int32),
                                        lanes])
            cur = plsc.load_gather(kout, [gvec, lanes])
            plsc.store_scatter(kout, [gvec, lanes], jnp.maximum(cur, v))

        pltpu.sync_copy(kout, outk_h.at[wid])

    return k(kin_p, gid3d)


def _k1_body(x_ref, w0_ref, wp_ref, a_ref, z_ref):
    xv = x_ref[...]
    y = _dotT(xv, w0_ref[...])
    z = _dotT(xv, wp_ref[...])
    n = xv.shape[0]
    one = jnp.ones((n, 1), jnp.float32)
    zpad = jnp.zeros((n, 15), jnp.float32)
    a_ref[...] = jnp.concatenate([y, one, zpad], axis=1)
    z_ref[...] = jnp.concatenate([z, one, zpad], axis=1)


def _mlp_tail(s1, zin, w1_ref, w2_ref, wo_ref, cv, last):
    t2 = _dotT(s1, w1_ref[...]) + cv[1:2, :]
    s2 = jnp.maximum(t2, 0.0)
    t3 = _dotT(s2, w2_ref[...]) + cv[2:3, :]
    s3 = jnp.maximum(t3, 0.0)
    h = jnp.maximum(s3 * cv[3:4, :] + cv[4:5, :], 0.0)
    o = _dotT(h, wo_ref[...])
    z = o if last else o[:, 32:80]
    zout = zin + jnp.concatenate(
        [z, jnp.zeros((z.shape[0], 16), jnp.float32)], axis=1)
    anext = None if last else o[:, :32]
    return anext, zout


def _k3_first_body(a_ref, p_ref, zin_ref, w1_ref, w2_ref, wo_ref, c_ref,
                   an_ref, zo_ref, rd_ref):
    agg = p_ref[0, :N] + p_ref[1, :N]
    cv = c_ref[...]
    rdeg = 1.0 / jnp.maximum(agg[:, 32:33], 1.0)
    s1 = jnp.maximum(a_ref[:, :32] + agg[:, :32] * rdeg + cv[0:1, :], 0.0)
    anext, zout = _mlp_tail(s1, zin_ref[...], w1_ref, w2_ref, wo_ref, cv,
                            last=False)
    an_ref[...] = anext
    zo_ref[...] = zout
    rd_ref[...] = rdeg


def _k3_mid_body(a_ref, p_ref, rd_ref, zin_ref, w1_ref, w2_ref, wo_ref,
                 c_ref, an_ref, zo_ref):
    agg = p_ref[0, :N] + p_ref[1, :N]
    cv = c_ref[...]
    s1 = jnp.maximum(a_ref[...] + agg * rd_ref[...] + cv[0:1, :], 0.0)
    anext, zout = _mlp_tail(s1, zin_ref[...], w1_ref, w2_ref, wo_ref, cv,
                            last=False)
    an_ref[...] = anext
    zo_ref[...] = zout


def _k3_last_body(a_ref, p_ref, rd_ref, zin_ref, w1_ref, w2_ref, wo_ref,
                  c_ref, zo_ref):
    agg = p_ref[0, :N] + p_ref[1, :N]
    cv = c_ref[...]
    s1 = jnp.maximum(a_ref[...] + agg * rd_ref[...] + cv[0:1, :], 0.0)
    _, zout = _mlp_tail(s1, zin_ref[...], w1_ref, w2_ref, wo_ref, cv,
                        last=True)
    zo_ref[:N] = zout
    zo_ref[N:] = jnp.zeros((NP - N, 64), jnp.float32)


def _k5_body(pp_ref, kp_ref, sb_ref, fw0_ref, c0_ref, fw1_ref, fb1_ref,
             out_ref):
    p = pp_ref[0] + pp_ref[1]
    counts = jnp.maximum(p[:, 48:49], 1.0)
    score = p[:, :48] / counts + sb_ref[...]
    km = jnp.max(kp_ref[...], axis=0)
    km = jnp.where(jnp.isfinite(km), km, 0.0)
    hc = jnp.concatenate([score, km], axis=1)
    sh = jnp.maximum(_dotT(hc, fw0_ref[...]) + c0_ref[...], 0.0)
    out_ref[...] = _dotT(sh, fw1_ref[...]) + fb1_ref[...]


def _tc(body, out_shape, *args):
    return pl.pallas_call(body, out_shape=out_shape,
                          compiler_params=_TC_PARAMS)(*args)


def _sds(shape):
    return jax.ShapeDtypeStruct(shape, jnp.float32)


def kernel(x, kinematics, params, edge_index, graph_ids):
    f32 = jnp.float32
    gin = params["gin"]

    def fold(l):
        lp = gin[l]
        g1, bb1 = lp["mlp_bn_g"][0], lp["mlp_bn_b"][0]
        g2, bb2 = lp["mlp_bn_g"][1], lp["mlp_bn_b"][1]
        ga, ba = lp["apply_bn_g"], lp["apply_bn_b"]
        w0 = lp["W"][0] * g1[:, None]
        c1 = lp["b"][0] * g1 + bb1
        w1 = lp["W"][1] * g2[:, None]
        c2 = lp["b"][1] * g2 + bb2
        w2 = lp["W"][2] * ga[:, None]
        c3 = lp["b"][2] * ga + ba
        return w0, w1, w2, c1, c2, c3, lp["out_bn_g"], lp["out_bn_b"]

    folded = [fold(l) for l in range(4)]

    a, zsum = _tc(_k1_body, (_sds((N, 48)), _sds((N, 64))),
                  x, folded[0][0], params["pred_W"][0])

    kin_p = jnp.concatenate(
        [kinematics, jnp.asarray(np.zeros((NP - N, 16), np.float32))], axis=0)
    gid3d = jnp.concatenate(
        [graph_ids.astype(jnp.int32),
         jnp.asarray(np.full((NP - N,), G, np.int32))]).reshape(
             NW, PRT // 80, 80)
    kin_part = _pool_kin(kin_p, gid3d)

    edges3d = edge_index.reshape(2, NW * CPT, CH)
    zeros48 = jnp.asarray(np.zeros((NACC, 48), np.float32))
    zeros32 = jnp.asarray(np.zeros((NACC, 32), np.float32))

    rdeg = None
    for l in range(4):
        width = 48 if l == 0 else 32
        p = _edge_agg(a, edges3d, zeros48 if l == 0 else zeros32, width)
        _, w1, w2, c1, c2, c3, go, bo = folded[l]
        if l < 3:
            wo = jnp.concatenate([folded[l + 1][0], params["pred_W"][l + 1]],
                                 axis=0)
        else:
            wo = params["pred_W"][4]
        cvec = jnp.stack([c1, c2, c3, go, bo])
        if l == 0:
            a, zsum, rdeg = _tc(
                _k3_first_body,
                (_sds((N, 32)), _sds((N, 64)), _sds((N, 1))),
                a, p, zsum, w1, w2, wo, cvec)
        elif l < 3:
            a, zsum = _tc(
                _k3_mid_body,
                (_sds((N, 32)), _sds((N, 64))),
                a, p, rdeg, zsum, w1, w2, wo, cvec)
        else:
            zsum_p = _tc(
                _k3_last_body,
                _sds((NP, 64)),
                a, p, rdeg, zsum, w1, w2, wo, cvec)

    pooled_p = _pool_sum(zsum_p, gid3d,
                         jnp.asarray(np.zeros((GP, 64), np.float32)))

    sumb = sum(params["pred_b"]).reshape(1, 48)
    fg = params["final_bn_g"][0]
    fw0 = params["final_W"][0] * fg[:, None]
    c0 = (params["final_b"][0] * fg + params["final_bn_b"][0]).reshape(1, 64)
    out = _tc(_k5_body, _sds((GP, 10)),
              pooled_p, kin_part, sumb, fw0, c0,
              params["final_W"][1], params["final_b"][1].reshape(1, 10))
    return out[:G]


# speedup vs baseline: 1.0144x; 1.0144x over previous
import functools

import numpy as np
import jax
import jax.numpy as jnp
from jax import lax
from jax.experimental import pallas as pl
from jax.experimental.pallas import tpu as pltpu
from jax.experimental.pallas import tpu_sc as plsc

N = 10000
E = 320000
G = 100
NC, NS, LANES = 2, 16, 16
NW = NC * NS
CH = 125
CPT = E // NW // CH
NBUF = 4
NACC = 10240
RPT = NACC // NS
NP = 10240
PRT = NP // NW
GP = 128

_MESH = plsc.VectorSubcoreMesh(
    core_axis_name="c", subcore_axis_name="s", num_cores=NC, num_subcores=NS
)

_SC_PARAMS = pltpu.CompilerParams(use_tc_tiling_on_sc=False,
                                  needs_layout_passes=False)
_TC_PARAMS = pltpu.CompilerParams(vmem_limit_bytes=100 * 1024 * 1024)


def _dotT(a, b):
    return lax.dot_general(a, b, (((1,), (1,)), ((), ())),
                           preferred_element_type=jnp.float32)


def _edge_agg(table, edges3d, zeros, width):
    @functools.partial(
        pl.kernel,
        out_type=jax.ShapeDtypeStruct((NC, NACC, width), jnp.float32),
        mesh=_MESH,
        scratch_types=[
            pltpu.VMEM((CPT, CH), jnp.int32),
            pltpu.VMEM((CPT, CH), jnp.int32),
        ] + [pltpu.VMEM((CH, width), jnp.float32) for _ in range(NBUF)]
          + [pltpu.VMEM_SHARED((NACC, width), jnp.float32)]
          + [pltpu.SemaphoreType.DMA for _ in range(NBUF)],
        compiler_params=_SC_PARAMS,
    )
    def k(table_h, edges_h, zeros_h, out_h, src_v, dst_v, *rest):
        bufs = rest[:NBUF]
        acc = rest[NBUF]
        gsem = rest[NBUF + 1:]
        c = lax.axis_index("c")
        s = lax.axis_index("s")
        wid = c * NS + s
        pltpu.sync_copy(zeros_h.at[pl.ds(s * RPT, RPT)],
                        acc.at[pl.ds(s * RPT, RPT)])
        pltpu.sync_copy(edges_h.at[0, pl.ds(wid * CPT, CPT)], src_v)
        pltpu.sync_copy(edges_h.at[1, pl.ds(wid * CPT, CPT)], dst_v)
        for b in range(NBUF):
            pltpu.async_copy(table_h.at[src_v.at[b]], bufs[b], gsem[b])
        plsc.subcore_barrier()

        @pl.loop(0, CPT, step=NBUF)
        def _(g):
            for b in range(NBUF):
                j = g + b
                pltpu.make_async_copy(table_h.at[src_v.at[j]], bufs[b],
                                      gsem[b]).wait()
                pltpu.sync_copy(bufs[b], acc.at[dst_v.at[j]], add=True)
                nxt = j + NBUF

                @pl.when(nxt < CPT)
                def _():
                    pltpu.async_copy(table_h.at[src_v.at[nxt]], bufs[b],
                                     gsem[b])

        plsc.subcore_barrier()
        pltpu.sync_copy(acc.at[pl.ds(s * RPT, RPT)],
                        out_h.at[c, pl.ds(s * RPT, RPT)])

    return k(table, edges3d, zeros)


def _pool_sum(zsum_p, gid3d, zeros_pool):
    @functools.partial(
        pl.kernel,
        out_type=jax.ShapeDtypeStruct((NC, GP, 64), jnp.float32),
        mesh=_MESH,
        scratch_types=[
            pltpu.VMEM((1, PRT // 80, 80), jnp.int32),
            pltpu.VMEM((PRT, 64), jnp.float32),
            pltpu.VMEM_SHARED((GP, 64), jnp.float32),
        ],
        compiler_params=_SC_PARAMS,
    )
    def k(zsum_h, gid_h, zeros_h, outp_h, gv, zbuf, pacc):
        c = lax.axis_index("c")
        s = lax.axis_index("s")
        wid = c * NS + s

        @pl.when(s == 0)
        def _():
            pltpu.sync_copy(zeros_h, pacc)

        pltpu.sync_copy(gid_h.at[pl.ds(wid, 1)], gv)
        pltpu.sync_copy(zsum_h.at[pl.ds(wid * PRT, PRT)], zbuf)
        plsc.subcore_barrier()
        for j in range(PRT // 80):
            pltpu.sync_copy(zbuf.at[pl.ds(j * 80, 80)], pacc.at[gv.at[0, j]],
                            add=True)
        plsc.subcore_barrier()

        @pl.when(s == 0)
        def _():
            pltpu.sync_copy(pacc, outp_h.at[c])

    return k(zsum_p, gid3d, zeros_pool)


def _pool_kin(kin_p, gid3d):
    @functools.partial(
        pl.kernel,
        out_type=jax.ShapeDtypeStruct((NW, GP, 16), jnp.float32),
        mesh=_MESH,
        scratch_types=[
            pltpu.VMEM((1, PRT // 80, 80), jnp.int32),
            pltpu.VMEM((PRT, 16), jnp.float32),
            pltpu.VMEM((GP, 16), jnp.float32),
        ],
        compiler_params=_SC_PARAMS,
    )
    def k(kin_h, gid_h, outk_h, gv, kbuf, kout):
        c = lax.axis_index("c")
        s = lax.axis_index("s")
        wid = c * NS + s
        pltpu.sync_copy(gid_h.at[pl.ds(wid, 1)], gv)
        pltpu.sync_copy(kin_h.at[pl.ds(wid * PRT, PRT)], kbuf)

        lanes = lax.iota(jnp.int32, LANES)
        neg = jnp.full((LANES,), -jnp.inf, jnp.float32)

        @pl.loop(0, GP)
        def _(i):
            plsc.store_scatter(kout, [jnp.full((LANES,), i, jnp.int32),
                                      lanes], neg)

        @pl.loop(0, PRT)
        def _(r):
            gvec = plsc.load_gather(
                gv, [jnp.zeros((LANES,), jnp.int32),
                     jnp.full((LANES,), r // 80, jnp.int32),
                     jnp.full((LANES,), r % 80, jnp.int32)])
            v = plsc.load_gather(kbuf, [jnp.full((LANES,), r, jnp.int32),
                                        lanes])
            cur = plsc.load_gather(kout, [gvec, lanes])
            plsc.store_scatter(kout, [gvec, lanes], jnp.maximum(cur, v))

        pltpu.sync_copy(kout, outk_h.at[wid])

    return k(kin_p, gid3d)


def _k1_body(x_ref, w0_ref, wp_ref, a_ref, z_ref):
    xv = x_ref[...]
    y = _dotT(xv, w0_ref[...])
    z = _dotT(xv, wp_ref[...])
    n = xv.shape[0]
    one = jnp.ones((n, 1), jnp.float32)
    zpad = jnp.zeros((n, 15), jnp.float32)
    a_ref[...] = jnp.concatenate([y, one, zpad], axis=1)
    z_ref[...] = jnp.concatenate([z, one, zpad], axis=1)


def _mlp_tail(s1, zin, w1_ref, w2_ref, wo_ref, cv, last):
    t2 = _dotT(s1, w1_ref[...]) + cv[1:2, :]
    s2 = jnp.maximum(t2, 0.0)
    t3 = _dotT(s2, w2_ref[...]) + cv[2:3, :]
    s3 = jnp.maximum(t3, 0.0)
    h = jnp.maximum(s3 * cv[3:4, :] + cv[4:5, :], 0.0)
    o = _dotT(h, wo_ref[...])
    z = o if last else o[:, 32:80]
    zout = zin + jnp.concatenate(
        [z, jnp.zeros((z.shape[0], 16), jnp.float32)], axis=1)
    anext = None if last else o[:, :32]
    return anext, zout


def _k3_first_body(a_ref, p_ref, zin_ref, w1_ref, w2_ref, wo_ref, c_ref,
                   an_ref, zo_ref, rd_ref):
    agg = p_ref[0, :N] + p_ref[1, :N]
    cv = c_ref[...]
    rdeg = 1.0 / jnp.maximum(agg[:, 32:33], 1.0)
    s1 = jnp.maximum(a_ref[:, :32] + agg[:, :32] * rdeg + cv[0:1, :], 0.0)
    anext, zout = _mlp_tail(s1, zin_ref[...], w1_ref, w2_ref, wo_ref, cv,
                            last=False)
    an_ref[...] = anext
    zo_ref[...] = zout
    rd_ref[...] = rdeg


def _k3_mid_body(a_ref, p_ref, rd_ref, zin_ref, w1_ref, w2_ref, wo_ref,
                 c_ref, an_ref, zo_ref):
    agg = p_ref[0, :N] + p_ref[1, :N]
    cv = c_ref[...]
    s1 = jnp.maximum(a_ref[...] + agg * rd_ref[...] + cv[0:1, :], 0.0)
    anext, zout = _mlp_tail(s1, zin_ref[...], w1_ref, w2_ref, wo_ref, cv,
                            last=False)
    an_ref[...] = anext
    zo_ref[...] = zout


def _k3_last_body(a_ref, p_ref, rd_ref, zin_ref, w1_ref, w2_ref, wo_ref,
                  c_ref, zo_ref):
    agg = p_ref[0, :N] + p_ref[1, :N]
    cv = c_ref[...]
    s1 = jnp.maximum(a_ref[...] + agg * rd_ref[...] + cv[0:1, :], 0.0)
    _, zout = _mlp_tail(s1, zin_ref[...], w1_ref, w2_ref, wo_ref, cv,
                        last=True)
    zo_ref[:N] = zout
    zo_ref[N:] = jnp.zeros((NP - N, 64), jnp.float32)


def _k5_body(pp_ref, kp_ref, sb_ref, fw0_ref, c0_ref, fw1_ref, fb1_ref,
             out_ref):
    p = pp_ref[0] + pp_ref[1]
    counts = jnp.maximum(p[:, 48:49], 1.0)
    score = p[:, :48] / counts + sb_ref[...]
    km = jnp.max(kp_ref[...], axis=0)
    km = jnp.where(jnp.isfinite(km), km, 0.0)
    hc = jnp.concatenate([score, km], axis=1)
    sh = jnp.maximum(_dotT(hc, fw0_ref[...]) + c0_ref[...], 0.0)
    out_ref[...] = _dotT(sh, fw1_ref[...]) + fb1_ref[...]


def _tc(body, out_shape, *args):
    return pl.pallas_call(body, out_shape=out_shape,
                          compiler_params=_TC_PARAMS)(*args)


def _sds(shape):
    return jax.ShapeDtypeStruct(shape, jnp.float32)


def kernel(x, kinematics, params, edge_index, graph_ids):
    f32 = jnp.float32
    gin = params["gin"]

    def fold(l):
        lp = gin[l]
        g1, bb1 = lp["mlp_bn_g"][0], lp["mlp_bn_b"][0]
        g2, bb2 = lp["mlp_bn_g"][1], lp["mlp_bn_b"][1]
        ga, ba = lp["apply_bn_g"], lp["apply_bn_b"]
        w0 = lp["W"][0] * g1[:, None]
        c1 = lp["b"][0] * g1 + bb1
        w1 = lp["W"][1] * g2[:, None]
        c2 = lp["b"][1] * g2 + bb2
        w2 = lp["W"][2] * ga[:, None]
        c3 = lp["b"][2] * ga + ba
        return w0, w1, w2, c1, c2, c3, lp["out_bn_g"], lp["out_bn_b"]

    folded = [fold(l) for l in range(4)]

    a, zsum = _tc(_k1_body, (_sds((N, 48)), _sds((N, 64))),
                  x, folded[0][0], params["pred_W"][0])

    kin_p = jnp.concatenate(
        [kinematics, jnp.asarray(np.zeros((NP - N, 16), np.float32))], axis=0)
    gid3d = jnp.concatenate(
        [graph_ids.astype(jnp.int32),
         jnp.asarray(np.full((NP - N,), G, np.int32))]).reshape(
             NW, PRT // 80, 80)
    kin_part = _pool_kin(kin_p, gid3d)

    edges3d = edge_index.reshape(2, NW * CPT, CH)
    zeros48 = jnp.asarray(np.zeros((NACC, 48), np.float32))
    zeros32 = jnp.asarray(np.zeros((NACC, 32), np.float32))

    rdeg = None
    for l in range(4):
        width = 48 if l == 0 else 32
        p = _edge_agg(a, edges3d, zeros48 if l == 0 else zeros32, width)
        _, w1, w2, c1, c2, c3, go, bo = folded[l]
        if l < 3:
            wo = jnp.concatenate([folded[l + 1][0], params["pred_W"][l + 1]],
                                 axis=0)
        else:
            wo = params["pred_W"][4]
        cvec = jnp.stack([c1, c2, c3, go, bo])
        if l == 0:
            a, zsum, rdeg = _tc(
                _k3_first_body,
                (_sds((N, 32)), _sds((N, 64)), _sds((N, 1))),
                a, p, zsum, w1, w2, wo, cvec)
        elif l < 3:
            a, zsum = _tc(
                _k3_mid_body,
                (_sds((N, 32)), _sds((N, 64))),
                a, p, rdeg, zsum, w1, w2, wo, cvec)
        else:
            zsum_p = _tc(
                _k3_last_body,
                _sds((NP, 64)),
                a, p, rdeg, zsum, w1, w2, wo, cvec)

    pooled_p = _pool_sum(zsum_p, gid3d,
                         jnp.asarray(np.zeros((GP, 64), np.float32)))

    sumb = sum(params["pred_b"]).reshape(1, 48)
    fg = params["final_bn_g"][0]
    fw0 = params["final_W"][0] * fg[:, None]
    c0 = (params["final_b"][0] * fg + params["final_bn_b"][0]).reshape(1, 64)
    out = _tc(_k5_body, _sds((GP, 10)),
              pooled_p, kin_part, sumb, fw0, c0,
              params["final_W"][1], params["final_b"][1].reshape(1, 10))
    return out[:G]


# speedup vs baseline: 1.1962x; 1.1792x over previous
import functools

import numpy as np
import jax
import jax.numpy as jnp
from jax import lax
from jax.experimental import pallas as pl
from jax.experimental.pallas import tpu as pltpu
from jax.experimental.pallas import tpu_sc as plsc

N = 10000
E = 320000
G = 100
NC, NS, LANES = 2, 16, 16
NW = NC * NS
CH = 125
CPT = E // NW // CH
NBUF = 4
NACC = 10240
RPT = NACC // NS
NP = 10240
PRT = NP // NW
GP = 128
QR = NACC // 4

_MESH = plsc.VectorSubcoreMesh(
    core_axis_name="c", subcore_axis_name="s", num_cores=NC, num_subcores=NS
)

_SC_PARAMS = pltpu.CompilerParams(use_tc_tiling_on_sc=False,
                                  needs_layout_passes=False)
_TC_PARAMS = pltpu.CompilerParams(vmem_limit_bytes=100 * 1024 * 1024)


def _dotT(a, b):
    return lax.dot_general(a, b, (((1,), (1,)), ((), ())),
                           preferred_element_type=jnp.float32)


def _dotN(a, b):
    return lax.dot_general(a, b, (((1,), (0,)), ((), ())),
                           preferred_element_type=jnp.float32)


def _edge_agg(table, edges3d, zeros, width):
    @functools.partial(
        pl.kernel,
        out_type=jax.ShapeDtypeStruct((NC, NACC, width), jnp.float32),
        mesh=_MESH,
        scratch_types=[
            pltpu.VMEM((CPT, CH), jnp.int32),
            pltpu.VMEM((CPT, CH), jnp.int32),
        ] + [pltpu.VMEM((CH, width), jnp.float32) for _ in range(NBUF)]
          + [pltpu.VMEM_SHARED((NACC, width), jnp.float32)]
          + [pltpu.SemaphoreType.DMA for _ in range(NBUF)],
        compiler_params=_SC_PARAMS,
    )
    def k(table_h, edges_h, zeros_h, out_h, src_v, dst_v, *rest):
        bufs = rest[:NBUF]
        acc = rest[NBUF]
        gsem = rest[NBUF + 1:]
        c = lax.axis_index("c")
        s = lax.axis_index("s")
        wid = c * NS + s
        pltpu.sync_copy(zeros_h.at[pl.ds(s * RPT, RPT)],
                        acc.at[pl.ds(s * RPT, RPT)])
        pltpu.sync_copy(edges_h.at[0, pl.ds(wid * CPT, CPT)], src_v)
        pltpu.sync_copy(edges_h.at[1, pl.ds(wid * CPT, CPT)], dst_v)
        for b in range(NBUF):
            pltpu.async_copy(table_h.at[src_v.at[b]], bufs[b], gsem[b])
        plsc.subcore_barrier()

        @pl.loop(0, CPT, step=NBUF)
        def _(g):
            for b in range(NBUF):
                j = g + b
                pltpu.make_async_copy(table_h.at[src_v.at[j]], bufs[b],
                                      gsem[b]).wait()
                pltpu.sync_copy(bufs[b], acc.at[dst_v.at[j]], add=True)
                nxt = j + NBUF

                @pl.when(nxt < CPT)
                def _():
                    pltpu.async_copy(table_h.at[src_v.at[nxt]], bufs[b],
                                     gsem[b])

        plsc.subcore_barrier()
        pltpu.sync_copy(acc.at[pl.ds(s * RPT, RPT)],
                        out_h.at[c, pl.ds(s * RPT, RPT)])

    return k(table, edges3d, zeros)


def _pool_sum(zsum_p, gid3d, zeros_pool):
    @functools.partial(
        pl.kernel,
        out_type=jax.ShapeDtypeStruct((NC, GP, 64), jnp.float32),
        mesh=_MESH,
        scratch_types=[
            pltpu.VMEM((1, PRT // 80, 80), jnp.int32),
            pltpu.VMEM((PRT, 64), jnp.float32),
            pltpu.VMEM_SHARED((GP, 64), jnp.float32),
        ],
        compiler_params=_SC_PARAMS,
    )
    def k(zsum_h, gid_h, zeros_h, outp_h, gv, zbuf, pacc):
        c = lax.axis_index("c")
        s = lax.axis_index("s")
        wid = c * NS + s

        @pl.when(s == 0)
        def _():
            pltpu.sync_copy(zeros_h, pacc)

        pltpu.sync_copy(gid_h.at[pl.ds(wid, 1)], gv)
        pltpu.sync_copy(zsum_h.at[pl.ds(wid * PRT, PRT)], zbuf)
        plsc.subcore_barrier()
        for j in range(PRT // 80):
            pltpu.sync_copy(zbuf.at[pl.ds(j * 80, 80)], pacc.at[gv.at[0, j]],
                            add=True)
        plsc.subcore_barrier()

        @pl.when(s == 0)
        def _():
            pltpu.sync_copy(pacc, outp_h.at[c])

    return k(zsum_p, gid3d, zeros_pool)


def _pool_kin(kin_p, gid3d):
    @functools.partial(
        pl.kernel,
        out_type=jax.ShapeDtypeStruct((NW, GP, 16), jnp.float32),
        mesh=_MESH,
        scratch_types=[
            pltpu.VMEM((1, PRT // 80, 80), jnp.int32),
            pltpu.VMEM((PRT, 16), jnp.float32),
            pltpu.VMEM((GP, 16), jnp.float32),
        ],
        compiler_params=_SC_PARAMS,
    )
    def k(kin_h, gid_h, outk_h, gv, kbuf, kout):
        c = lax.axis_index("c")
        s = lax.axis_index("s")
        wid = c * NS + s
        pltpu.sync_copy(gid_h.at[pl.ds(wid, 1)], gv)
        pltpu.sync_copy(kin_h.at[pl.ds(wid * PRT, PRT)], kbuf)

        lanes = lax.iota(jnp.int32, LANES)
        neg = jnp.full((LANES,), -jnp.inf, jnp.float32)

        @pl.loop(0, GP)
        def _(i):
            plsc.store_scatter(kout, [jnp.full((LANES,), i, jnp.int32),
                                      lanes], neg)

        @pl.loop(0, PRT)
        def _(r):
            gvec = plsc.load_gather(
                gv, [jnp.zeros((LANES,), jnp.int32),
                     jnp.full((LANES,), r // 80, jnp.int32),
                     jnp.full((LANES,), r % 80, jnp.int32)])
            v = plsc.load_gather(kbuf, [jnp.full((LANES,), r, jnp.int32),
                                        lanes])
            cur = plsc.load_gather(kout, [gvec, lanes])
            plsc.store_scatter(kout, [gvec, lanes], jnp.maximum(cur, v))

        pltpu.sync_copy(kout, outk_h.at[wid])

    return k(kin_p, gid3d)


def _k1_body(x_ref, w0_ref, wp_ref, a_ref, z_ref):
    xv = x_ref[...]
    y = _dotT(xv, w0_ref[...])
    z = _dotT(xv, wp_ref[...])
    n = xv.shape[0]
    one = jnp.ones((n, 1), jnp.float32)
    zpad = jnp.zeros((n, 15), jnp.float32)
    a_ref[...] = jnp.concatenate([y, one, zpad], axis=1)
    z_ref[...] = jnp.concatenate([z, one, zpad], axis=1)


def _pack4(v):
    return jnp.concatenate([v[q * QR:(q + 1) * QR] for q in range(4)], axis=1)


def _k3_first_body(a_ref, p_ref, zin_ref, w1_ref, w2_ref, wo_ref, c_ref,
                   an_ref, zo_ref, rd_ref):
    psum = p_ref[0] + p_ref[1]
    cv = c_ref[...]
    rdeg_full = 1.0 / jnp.maximum(psum[:, 32:33], 1.0)
    agg = psum[:N]
    s1 = jnp.maximum(a_ref[:, :32] + agg[:, :32] * rdeg_full[:N] + cv[0:1, :],
                     0.0)
    t2 = jnp.maximum(_dotT(s1, w1_ref[...]) + cv[1:2, :], 0.0)
    t3 = jnp.maximum(_dotT(t2, w2_ref[...]) + cv[2:3, :], 0.0)
    h = jnp.maximum(t3 * cv[3:4, :] + cv[4:5, :], 0.0)
    o = _dotT(h, wo_ref[...])
    a32p = jnp.concatenate(
        [o[:, :32], jnp.zeros((NACC - N, 32), jnp.float32)], axis=0)
    an_ref[...] = _pack4(a32p)
    zo_ref[...] = zin_ref[...] + jnp.concatenate(
        [o[:, 32:80], jnp.zeros((N, 16), jnp.float32)], axis=1)
    rd_ref[...] = _pack4(rdeg_full)


def _packed_head(a_ref, p_ref, rd_ref, c_ref, w1_ref, w2_ref):
    cv = c_ref[...]
    agg = p_ref[0] + p_ref[1]
    rd = rd_ref[...]
    rdrep = jnp.concatenate(
        [jnp.broadcast_to(rd[:, q:q + 1], (QR, 32)) for q in range(4)], axis=1)
    s1 = jnp.maximum(a_ref[...] + agg * rdrep + cv[0:1, :], 0.0)
    t2 = jnp.maximum(_dotN(s1, w1_ref[...]) + cv[1:2, :], 0.0)
    t3 = jnp.maximum(_dotN(t2, w2_ref[...]) + cv[2:3, :], 0.0)
    return jnp.maximum(t3 * cv[3:4, :] + cv[4:5, :], 0.0)


def _k3_mid_body(a_ref, p_ref, rd_ref, zin_ref, w1_ref, w2_ref, wo_ref,
                 c_ref, an_ref, zo_ref):
    h = _packed_head(a_ref, p_ref, rd_ref, c_ref, w1_ref, w2_ref)
    o = _dotN(h, wo_ref[...])
    an_ref[...] = jnp.concatenate(
        [o[:, q * 80:q * 80 + 32] for q in range(4)], axis=1)
    zn = jnp.concatenate(
        [o[:, q * 80 + 32:q * 80 + 80] for q in range(4)], axis=0)[:N]
    zo_ref[...] = zin_ref[...] + jnp.concatenate(
        [zn, jnp.zeros((N, 16), jnp.float32)], axis=1)


def _k3_last_body(a_ref, p_ref, rd_ref, zin_ref, w1_ref, w2_ref, wo_ref,
                  c_ref, zo_ref):
    h = _packed_head(a_ref, p_ref, rd_ref, c_ref, w1_ref, w2_ref)
    o = _dotN(h, wo_ref[...])
    zn = jnp.concatenate(
        [o[:, q * 48:(q + 1) * 48] for q in range(4)], axis=0)[:N]
    zo_ref[:N] = zin_ref[...] + jnp.concatenate(
        [zn, jnp.zeros((N, 16), jnp.float32)], axis=1)
    zo_ref[N:] = jnp.zeros((NP - N, 64), jnp.float32)


def _k5_body(pp_ref, kp_ref, sb_ref, fw0_ref, c0_ref, fw1_ref, fb1_ref,
             out_ref):
    p = pp_ref[0] + pp_ref[1]
    counts = jnp.maximum(p[:, 48:49], 1.0)
    score = p[:, :48] / counts + sb_ref[...]
    km = jnp.max(kp_ref[...], axis=0)
    km = jnp.where(jnp.isfinite(km), km, 0.0)
    hc = jnp.concatenate([score, km], axis=1)
    sh = jnp.maximum(_dotT(hc, fw0_ref[...]) + c0_ref[...], 0.0)
    out_ref[...] = _dotT(sh, fw1_ref[...]) + fb1_ref[...]


def _tc(body, out_shape, *args):
    return pl.pallas_call(body, out_shape=out_shape,
                          compiler_params=_TC_PARAMS)(*args)


def _sds(shape):
    return jax.ShapeDtypeStruct(shape, jnp.float32)


def kernel(x, kinematics, params, edge_index, graph_ids):
    f32 = jnp.float32
    gin = params["gin"]

    def fold(l):
        lp = gin[l]
        g1, bb1 = lp["mlp_bn_g"][0], lp["mlp_bn_b"][0]
        g2, bb2 = lp["mlp_bn_g"][1], lp["mlp_bn_b"][1]
        ga, ba = lp["apply_bn_g"], lp["apply_bn_b"]
        w0 = lp["W"][0] * g1[:, None]
        c1 = lp["b"][0] * g1 + bb1
        w1 = lp["W"][1] * g2[:, None]
        c2 = lp["b"][1] * g2 + bb2
        w2 = lp["W"][2] * ga[:, None]
        c3 = lp["b"][2] * ga + ba
        return w0, w1, w2, c1, c2, c3, lp["out_bn_g"], lp["out_bn_b"]

    folded = [fold(l) for l in range(4)]

    a, zsum = _tc(_k1_body, (_sds((N, 48)), _sds((N, 64))),
                  x, folded[0][0], params["pred_W"][0])

    kin_p = jnp.concatenate(
        [kinematics, jnp.asarray(np.zeros((NP - N, 16), np.float32))], axis=0)
    gid3d = jnp.concatenate(
        [graph_ids.astype(jnp.int32),
         jnp.asarray(np.full((NP - N,), G, np.int32))]).reshape(
             NW, PRT // 80, 80)
    kin_part = _pool_kin(kin_p, gid3d)

    edges3d = edge_index.reshape(2, NW * CPT, CH)
    edges_pk = ((edge_index % QR) * 4 + edge_index // QR).reshape(
        2, NW * CPT, CH)
    zeros48 = jnp.asarray(np.zeros((NACC, 48), np.float32))
    zeros32 = jnp.asarray(np.zeros((NACC, 32), np.float32))
    eye4 = jnp.asarray(np.eye(4, dtype=np.float32))

    rdeg = None
    for l in range(4):
        _, w1, w2, c1, c2, c3, go, bo = folded[l]
        if l < 3:
            wo = jnp.concatenate([folded[l + 1][0], params["pred_W"][l + 1]],
                                 axis=0)
        else:
            wo = params["pred_W"][4]
        cvec = jnp.stack([c1, c2, c3, go, bo])
        if l == 0:
            p = _edge_agg(a, edges3d, zeros48, 48)
            a, zsum, rdeg = _tc(
                _k3_first_body,
                (_sds((QR, 128)), _sds((N, 64)), _sds((QR, 4))),
                a, p, zsum, w1, w2, wo, cvec)
        else:
            p = _edge_agg(a.reshape(NACC, 32), edges_pk, zeros32, 32)
            p = p.reshape(2, QR, 128)
            w1b = jnp.kron(eye4, w1.T)
            w2b = jnp.kron(eye4, w2.T)
            wob = jnp.kron(eye4, wo.T)
            cvt = jnp.tile(cvec, (1, 4))
            if l < 3:
                a, zsum = _tc(
                    _k3_mid_body,
                    (_sds((QR, 128)), _sds((N, 64))),
                    a, p, rdeg, zsum, w1b, w2b, wob, cvt)
            else:
                zsum_p = _tc(
                    _k3_last_body,
                    _sds((NP, 64)),
                    a, p, rdeg, zsum, w1b, w2b, wob, cvt)

    pooled_p = _pool_sum(zsum_p, gid3d,
                         jnp.asarray(np.zeros((GP, 64), np.float32)))

    sumb = sum(params["pred_b"]).reshape(1, 48)
    fg = params["final_bn_g"][0]
    fw0 = params["final_W"][0] * fg[:, None]
    c0 = (params["final_b"][0] * fg + params["final_bn_b"][0]).reshape(1, 64)
    out = _tc(_k5_body, _sds((GP, 10)),
              pooled_p, kin_part, sumb, fw0, c0,
              params["final_W"][1], params["final_b"][1].reshape(1, 10))
    return out[:G]
